# baseline (speedup 1.0000x reference)
import functools

import numpy as np
import jax
import jax.numpy as jnp
from jax import lax
from jax.experimental import pallas as pl
from jax.experimental.pallas import tpu as pltpu
from jax.experimental.pallas import tpu_sc as plsc

_HOP_SIZE = 512
_SAMPLE_RATE = 44100
_ENC_SAMPLE_RATE = 16000
_ENC_HOP_SIZE = 320
_N_FRAMES = 14000

_NUM_WORKERS = 32
_ROWS_PER_WORKER = 56
_CHUNKS = (32, 24)
_F_SC = _NUM_WORKERS * _ROWS_PER_WORKER

_BF = 112
_BI = 80
_NB = (_N_FRAMES - _F_SC) // _BF


def _np_index(n_frames: int, num_src_rows: int) -> np.ndarray:
    ratio = _HOP_SIZE / _SAMPLE_RATE / (_ENC_HOP_SIZE / _ENC_SAMPLE_RATE)
    idx = np.round(ratio * np.arange(n_frames)).astype(np.int64)
    return np.minimum(idx, num_src_rows - 1).astype(np.int32)


def _sc_body(units_h, idx_h, out_h, idx_v, buf0, buf1,
             sem_in0, sem_in1, sem_out0, sem_out1):
    info = plsc.get_sparse_core_info()
    wid = lax.axis_index("s") * info.num_cores + lax.axis_index("c")
    base = wid * _ROWS_PER_WORKER
    bufs = (buf0, buf1)
    sin = (sem_in0, sem_in1)
    sout = (sem_out0, sem_out1)
    offs = [0]
    for c in _CHUNKS:
        offs.append(offs[-1] + c)
    pltpu.sync_copy(idx_h.at[wid], idx_v)

    def gather(c):
        return pltpu.async_copy(
            units_h.at[idx_v.at[0, pl.ds(offs[c], _CHUNKS[c])]],
            bufs[c % 2].at[pl.ds(0, _CHUNKS[c])], sin[c % 2])

    def put(c):
        return pltpu.async_copy(
            bufs[c % 2].at[pl.ds(0, _CHUNKS[c])],
            out_h.at[pl.ds(base + offs[c], _CHUNKS[c])], sout[c % 2])

    n = len(_CHUNKS)
    hout = [None] * n
    hin = [None] * n
    hin[0] = gather(0)
    for c in range(n):
        if c + 1 < n:
            if c >= 1:
                hout[c - 1].wait()
            hin[c + 1] = gather(c + 1)
        hin[c].wait()
        hout[c] = put(c)
    for c in range(max(0, n - 2), n):
        hout[c].wait()


def _sc_gather(table, idx_np):
    feat = table.shape[1]
    idx = jnp.asarray(idx_np[:_F_SC].reshape(_NUM_WORKERS, 1, _ROWS_PER_WORKER))
    mesh = plsc.VectorSubcoreMesh(core_axis_name="c", subcore_axis_name="s")
    max_chunk = max(_CHUNKS)
    run = pl.kernel(
        _sc_body,
        out_type=jax.ShapeDtypeStruct((_F_SC, feat), jnp.float32),
        mesh=mesh,
        scratch_types=[
            pltpu.VMEM((1, _ROWS_PER_WORKER), jnp.int32),
            pltpu.VMEM((max_chunk, feat), jnp.float32),
            pltpu.VMEM((max_chunk, feat), jnp.float32),
            pltpu.SemaphoreType.DMA,
            pltpu.SemaphoreType.DMA,
            pltpu.SemaphoreType.DMA,
            pltpu.SemaphoreType.DMA,
        ],
    )
    return run(table, idx)


def _tc_body(starts_s, table_any, out_ref, buf, sem):
    b = pl.program_id(0)

    def span_copy(bb, slot):
        start = pl.multiple_of(starts_s[bb], 8)
        return pltpu.make_async_copy(
            table_any.at[pl.ds(start, _BI)], buf.at[slot], sem.at[slot])

    @pl.when(b == 0)
    def _():
        span_copy(0, 0).start()

    @pl.when(b + 1 < _NB)
    def _():
        span_copy(b + 1, (b + 1) % 2).start()

    span_copy(b, b % 2).wait()

    f0 = _F_SC + b * _BF
    s = starts_s[b]
    f2d = f0 + lax.broadcasted_iota(jnp.int32, (_BF, _BI), 0)
    r2d = s + lax.broadcasted_iota(jnp.int32, (_BF, _BI), 1)
    lo = lax.shift_right_arithmetic(882 * r2d + 71, 9)
    hi = lax.shift_right_arithmetic(882 * r2d + 953, 9)
    p = jnp.where((f2d >= lo) & (f2d < hi), 1.0, 0.0).astype(jnp.float32)
    out_ref[...] = jnp.dot(p, buf[b % 2], preferred_element_type=jnp.float32)


def _tc_expand(table, idx_np):
    num_src_rows, feat = table.shape
    f0s = _F_SC + np.arange(_NB) * _BF
    starts = (idx_np[f0s] // 8) * 8
    assert int((idx_np[f0s + _BF - 1] - starts).max()) < _BI
    assert int(starts.max()) + _BI <= num_src_rows
    assert _F_SC % _BF == 0

    return pl.pallas_call(
        _tc_body,
        grid=(_NB,),
        in_specs=[
            pl.BlockSpec(memory_space=pltpu.SMEM),
            pl.BlockSpec(memory_space=pl.ANY),
        ],
        out_specs=pl.BlockSpec((_BF, feat), lambda b: (b + _F_SC // _BF, 0)),
        out_shape=jax.ShapeDtypeStruct((_N_FRAMES, feat), jnp.float32),
        scratch_shapes=[
            pltpu.VMEM((2, _BI, feat), jnp.float32),
            pltpu.SemaphoreType.DMA((2,)),
        ],
    )(jnp.asarray(starts), table)


def kernel(units, n_frames):
    del n_frames
    _, num_src_rows, feat = units.shape
    table = units.reshape(num_src_rows, feat)
    idx_np = _np_index(_N_FRAMES, num_src_rows)
    sc_out = _sc_gather(table, idx_np)
    tc_out = _tc_expand(table, idx_np)
    out = lax.dynamic_update_slice(tc_out, sc_out, (0, 0))
    return out[None]


# speedup vs baseline: 1.7464x; 1.7464x over previous
import functools

import numpy as np
import jax
import jax.numpy as jnp
from jax import lax
from jax.experimental import pallas as pl
from jax.experimental.pallas import tpu as pltpu
from jax.experimental.pallas import tpu_sc as plsc

_HOP_SIZE = 512
_SAMPLE_RATE = 44100
_ENC_SAMPLE_RATE = 16000
_ENC_HOP_SIZE = 320
_N_FRAMES = 14000

_NUM_WORKERS = 32
_SC_SLAB = 40
_F_SC = 1120

_BF = 560
_BI = 336
_NB = (_N_FRAMES - _F_SC) // _BF


def _np_index(n_frames: int, num_src_rows: int) -> np.ndarray:
    ratio = _HOP_SIZE / _SAMPLE_RATE / (_ENC_HOP_SIZE / _ENC_SAMPLE_RATE)
    idx = np.round(ratio * np.arange(n_frames)).astype(np.int64)
    return np.minimum(idx, num_src_rows - 1).astype(np.int32)


def _sc_slab_bases() -> list[int]:
    return [((35 * w) >> 3) << 3 for w in range(_NUM_WORKERS)]


def _sc_body(units_h, idx_h, out_h, idx_v, buf, sem_in, sem_out):
    info = plsc.get_sparse_core_info()
    wid = lax.axis_index("s") * info.num_cores + lax.axis_index("c")
    base = pl.multiple_of(
        lax.shift_left(lax.shift_right_logical(35 * wid, 3), 3), 8)
    pltpu.sync_copy(idx_h.at[wid], idx_v)
    pltpu.async_copy(units_h.at[idx_v.at[0]], buf, sem_in).wait()
    pltpu.async_copy(buf, out_h.at[pl.ds(base, _SC_SLAB)], sem_out).wait()


def _sc_gather(table, idx_np):
    feat = table.shape[1]
    idx_slabs = np.stack(
        [idx_np[b:b + _SC_SLAB] for b in _sc_slab_bases()])[:, None, :]
    mesh = plsc.VectorSubcoreMesh(core_axis_name="c", subcore_axis_name="s")
    run = pl.kernel(
        _sc_body,
        out_type=jax.ShapeDtypeStruct((_F_SC, feat), jnp.float32),
        mesh=mesh,
        scratch_types=[
            pltpu.VMEM((1, _SC_SLAB), jnp.int32),
            pltpu.VMEM((_SC_SLAB, feat), jnp.float32),
            pltpu.SemaphoreType.DMA,
            pltpu.SemaphoreType.DMA,
        ],
    )
    return run(table, jnp.asarray(idx_slabs))


def _tc_body(starts_s, table_any, out_ref, buf, sem):
    b = pl.program_id(0)

    def span_copy(bb, slot):
        start = pl.multiple_of(starts_s[bb], 8)
        return pltpu.make_async_copy(
            table_any.at[pl.ds(start, _BI)], buf.at[slot], sem.at[slot])

    @pl.when(b == 0)
    def _():
        span_copy(0, 0).start()

    @pl.when(b + 1 < _NB)
    def _():
        span_copy(b + 1, (b + 1) % 2).start()

    span_copy(b, b % 2).wait()

    f0 = _F_SC + b * _BF
    s = starts_s[b]
    f2d = f0 + lax.broadcasted_iota(jnp.int32, (_BF, _BI), 0)
    r2d = s + lax.broadcasted_iota(jnp.int32, (_BF, _BI), 1)
    lo = lax.shift_right_arithmetic(882 * r2d + 71, 9)
    hi = lax.shift_right_arithmetic(882 * r2d + 953, 9)
    p = jnp.where((f2d >= lo) & (f2d < hi), 1.0, 0.0).astype(jnp.float32)
    out_ref[...] = jnp.dot(p, buf[b % 2], preferred_element_type=jnp.float32)


def _tc_expand(table, idx_np):
    num_src_rows, feat = table.shape
    f0s = _F_SC + np.arange(_NB) * _BF
    starts = (idx_np[f0s] // 8) * 8
    assert int((idx_np[f0s + _BF - 1] - starts).max()) < _BI
    assert int(starts.max()) + _BI <= num_src_rows
    assert _F_SC % _BF == 0

    return pl.pallas_call(
        _tc_body,
        grid=(_NB,),
        in_specs=[
            pl.BlockSpec(memory_space=pltpu.SMEM),
            pl.BlockSpec(memory_space=pl.ANY),
        ],
        out_specs=pl.BlockSpec((_BF, feat), lambda b: (b + _F_SC // _BF, 0)),
        out_shape=jax.ShapeDtypeStruct((_N_FRAMES, feat), jnp.float32),
        scratch_shapes=[
            pltpu.VMEM((2, _BI, feat), jnp.float32),
            pltpu.SemaphoreType.DMA((2,)),
        ],
    )(jnp.asarray(starts), table)


def kernel(units, n_frames):
    del n_frames
    _, num_src_rows, feat = units.shape
    table = units.reshape(num_src_rows, feat)
    idx_np = _np_index(_N_FRAMES, num_src_rows)
    sc_out = _sc_gather(table, idx_np)
    tc_out = _tc_expand(table, idx_np)
    out = lax.dynamic_update_slice(tc_out, sc_out, (0, 0))
    return out[None]


# speedup vs baseline: 1.7618x; 1.0088x over previous
import functools

import numpy as np
import jax
import jax.numpy as jnp
from jax import lax
from jax.experimental import pallas as pl
from jax.experimental.pallas import tpu as pltpu
from jax.experimental.pallas import tpu_sc as plsc

_HOP_SIZE = 512
_SAMPLE_RATE = 44100
_ENC_SAMPLE_RATE = 16000
_ENC_HOP_SIZE = 320
_N_FRAMES = 14000

_NUM_WORKERS = 32
_SC_SLAB = 24
_F_SC = 560

_BF = 560
_BI = 336
_NB = (_N_FRAMES - _F_SC) // _BF


def _np_index(n_frames: int, num_src_rows: int) -> np.ndarray:
    ratio = _HOP_SIZE / _SAMPLE_RATE / (_ENC_HOP_SIZE / _ENC_SAMPLE_RATE)
    idx = np.round(ratio * np.arange(n_frames)).astype(np.int64)
    return np.minimum(idx, num_src_rows - 1).astype(np.int32)


def _sc_slab_bases() -> list[int]:
    return [(_F_SC * w // _NUM_WORKERS // 8) * 8 for w in range(_NUM_WORKERS)]


def _sc_body(units_h, idx_h, out_h, idx_v, buf, sem_in, sem_out):
    info = plsc.get_sparse_core_info()
    wid = lax.axis_index("s") * info.num_cores + lax.axis_index("c")
    base = pl.multiple_of(
        lax.shift_left(lax.shift_right_logical(35 * wid, 4), 3), 8)
    pltpu.sync_copy(idx_h.at[wid], idx_v)
    pltpu.async_copy(units_h.at[idx_v.at[0]], buf, sem_in).wait()
    pltpu.async_copy(buf, out_h.at[pl.ds(base, _SC_SLAB)], sem_out).wait()


def _sc_gather(table, idx_np):
    feat = table.shape[1]
    idx_slabs = np.stack(
        [idx_np[b:b + _SC_SLAB] for b in _sc_slab_bases()])[:, None, :]
    mesh = plsc.VectorSubcoreMesh(core_axis_name="c", subcore_axis_name="s")
    run = pl.kernel(
        _sc_body,
        out_type=jax.ShapeDtypeStruct((_F_SC, feat), jnp.float32),
        mesh=mesh,
        scratch_types=[
            pltpu.VMEM((1, _SC_SLAB), jnp.int32),
            pltpu.VMEM((_SC_SLAB, feat), jnp.float32),
            pltpu.SemaphoreType.DMA,
            pltpu.SemaphoreType.DMA,
        ],
    )
    return run(table, jnp.asarray(idx_slabs))


def _tc_body(starts_s, table_any, out_ref, buf, sem):
    b = pl.program_id(0)

    def span_copy(bb, slot):
        start = pl.multiple_of(starts_s[bb], 8)
        return pltpu.make_async_copy(
            table_any.at[pl.ds(start, _BI)], buf.at[slot], sem.at[slot])

    @pl.when(b == 0)
    def _():
        span_copy(0, 0).start()

    @pl.when(b + 1 < _NB)
    def _():
        span_copy(b + 1, (b + 1) % 2).start()

    span_copy(b, b % 2).wait()

    f0 = _F_SC + b * _BF
    s = starts_s[b]
    f2d = f0 + lax.broadcasted_iota(jnp.int32, (_BF, _BI), 0)
    r2d = s + lax.broadcasted_iota(jnp.int32, (_BF, _BI), 1)
    lo = lax.shift_right_arithmetic(882 * r2d + 71, 9)
    hi = lax.shift_right_arithmetic(882 * r2d + 953, 9)
    p = jnp.where((f2d >= lo) & (f2d < hi), 1.0, 0.0).astype(jnp.float32)
    out_ref[...] = jnp.dot(p, buf[b % 2], preferred_element_type=jnp.float32)


def _tc_expand(table, idx_np):
    num_src_rows, feat = table.shape
    f0s = _F_SC + np.arange(_NB) * _BF
    starts = (idx_np[f0s] // 8) * 8
    assert int((idx_np[f0s + _BF - 1] - starts).max()) < _BI
    assert int(starts.max()) + _BI <= num_src_rows
    assert _F_SC % _BF == 0

    return pl.pallas_call(
        _tc_body,
        grid=(_NB,),
        in_specs=[
            pl.BlockSpec(memory_space=pltpu.SMEM),
            pl.BlockSpec(memory_space=pl.ANY),
        ],
        out_specs=pl.BlockSpec((_BF, feat), lambda b: (b + _F_SC // _BF, 0)),
        out_shape=jax.ShapeDtypeStruct((_N_FRAMES, feat), jnp.float32),
        scratch_shapes=[
            pltpu.VMEM((2, _BI, feat), jnp.float32),
            pltpu.SemaphoreType.DMA((2,)),
        ],
    )(jnp.asarray(starts), table)


def kernel(units, n_frames):
    del n_frames
    _, num_src_rows, feat = units.shape
    table = units.reshape(num_src_rows, feat)
    idx_np = _np_index(_N_FRAMES, num_src_rows)
    sc_out = _sc_gather(table, idx_np)
    tc_out = _tc_expand(table, idx_np)
    out = lax.dynamic_update_slice(tc_out, sc_out, (0, 0))
    return out[None]
